# baseline (speedup 1.0000x reference)
import functools

import jax
import jax.numpy as jnp
from jax import lax
from jax.experimental import pallas as pl
from jax.experimental.pallas import tpu as pltpu

_ROWS = 8


def _body(x_hbm, probs_ref, tok_ref, buf, sems):
    i = pl.program_id(0)
    n = pl.num_programs(0)
    l = x_hbm.shape[1]
    slot = lax.rem(i, 2)
    nslot = lax.rem(i + 1, 2)

    def start_block(blk, bslot):
        for r in range(_ROWS):
            pltpu.make_async_copy(
                x_hbm.at[blk * _ROWS + r, l - 1],
                buf.at[bslot, r],
                sems.at[bslot],
            ).start()

    def wait_block(blk, bslot):
        for r in range(_ROWS):
            pltpu.make_async_copy(
                x_hbm.at[blk * _ROWS + r, l - 1],
                buf.at[bslot, r],
                sems.at[bslot],
            ).wait()

    @pl.when(i == 0)
    def _():
        start_block(0, 0)

    @pl.when(i + 1 < n)
    def _():
        start_block(i + 1, nslot)

    wait_block(i, slot)

    x = buf[slot]
    r, v = x.shape
    m = jnp.max(x, axis=-1, keepdims=True)
    e = jnp.exp(x - m)
    s = jnp.sum(e, axis=-1, keepdims=True)
    probs_ref[...] = e * (1.0 / s)
    idx = lax.broadcasted_iota(jnp.int32, (r, v), 1)
    cand = jnp.where(x == m, idx, v)
    tok_ref[...] = jnp.min(cand, axis=-1, keepdims=True)


@jax.jit
def kernel(logits):
    b, l, v = logits.shape
    grid = (b // _ROWS,)
    probs, tok = pl.pallas_call(
        _body,
        grid=grid,
        in_specs=[pl.BlockSpec(memory_space=pltpu.MemorySpace.HBM)],
        out_specs=[
            pl.BlockSpec((_ROWS, v), lambda i: (i, 0)),
            pl.BlockSpec((_ROWS, 1), lambda i: (i, 0)),
        ],
        out_shape=[
            jax.ShapeDtypeStruct((b, v), jnp.float32),
            jax.ShapeDtypeStruct((b, 1), jnp.int32),
        ],
        scratch_shapes=[
            pltpu.VMEM((2, _ROWS, v), jnp.float32),
            pltpu.SemaphoreType.DMA((2,)),
        ],
    )(logits)
    return (tok.reshape(b), probs)


# speedup vs baseline: 2.5020x; 2.5020x over previous
import jax
import jax.numpy as jnp
from jax import lax
from jax.experimental import pallas as pl

_ROWS = 8


def _body(x_ref, probs_ref, tok_ref):
    x = x_ref[...]
    r, v = x.shape
    m = jnp.max(x, axis=-1, keepdims=True)
    e = jnp.exp(x - m)
    s = jnp.sum(e, axis=-1, keepdims=True)
    probs_ref[...] = e * (1.0 / s)
    idx = lax.broadcasted_iota(jnp.int32, (r, v), 1)
    cand = jnp.where(x == m, idx, v)
    tok_ref[...] = jnp.min(cand, axis=-1, keepdims=True)


@jax.jit
def kernel(logits):
    b, l, v = logits.shape
    last = logits[:, l - 1]
    grid = (b // _ROWS,)
    probs, tok = pl.pallas_call(
        _body,
        grid=grid,
        in_specs=[pl.BlockSpec((_ROWS, v), lambda i: (i, 0))],
        out_specs=[
            pl.BlockSpec((_ROWS, v), lambda i: (i, 0)),
            pl.BlockSpec((_ROWS, 1), lambda i: (i, 0)),
        ],
        out_shape=[
            jax.ShapeDtypeStruct((b, v), jnp.float32),
            jax.ShapeDtypeStruct((b, 1), jnp.int32),
        ],
    )(last)
    return (tok.reshape(b), probs)
